# Initial kernel scaffold; baseline (speedup 1.0000x reference)
import jax
import jax.numpy as jnp
from jax.experimental import pallas as pl


def kernel(x, table, W, b):
    raise NotImplementedError("write your pallas kernel here")


# speedup vs baseline: 2.8240x; 2.8240x over previous
import functools

import jax
import jax.numpy as jnp
from jax import lax
from jax.experimental import pallas as pl
from jax.experimental.pallas import tpu as pltpu
import jax.experimental.pallas.tpu_sc as plsc

_VOCAB = 1000000
_DIM = 64
_SEQ = 200
_BATCH = 16384


_BLK = 8192
_GRID = -(-_VOCAB // _BLK)


def _matvec_body(tb_ref, w_ref, b_ref, out_ref):
    tb = tb_ref[...]
    w = w_ref[...]
    z = jax.lax.dot_general(w, tb, (((1,), (1,)), ((), ())),
                            preferred_element_type=jnp.float32)
    out_ref[...] = z[0] + b_ref[0]


def _matvec(table, w_row, b):
    return pl.pallas_call(
        _matvec_body,
        grid=(_GRID,),
        in_specs=[
            pl.BlockSpec((_BLK, _DIM), lambda i: (i, 0)),
            pl.BlockSpec((1, _DIM), lambda i: (0, 0)),
            pl.BlockSpec(memory_space=pltpu.SMEM),
        ],
        out_specs=pl.BlockSpec((_BLK,), lambda i: (i,)),
        out_shape=jax.ShapeDtypeStruct((_VOCAB,), jnp.float32),
    )(table, w_row, b)


_NC, _NS, _L = 2, 16, 16
_NW = _NC * _NS
_BPW = _BATCH // _NW
_EPC = 64
_CH = _EPC * _SEQ
_NCHUNK = _BPW // _EPC
_NG = _EPC // _L


@functools.partial(
    pl.kernel,
    out_type=jax.ShapeDtypeStruct((_BATCH,), jnp.float32),
    mesh=plsc.VectorSubcoreMesh(core_axis_name="c", subcore_axis_name="s"),
    scratch_types=[
        pltpu.VMEM((_CH,), jnp.int32),
        pltpu.VMEM((_CH,), jnp.float32),
        pltpu.VMEM((_BPW,), jnp.float32),
        pltpu.SemaphoreType.DMA,
    ],
)
def _sc_pool(t_hbm, xt_hbm, out_hbm, idx_v, vals_v, outb_v, sem):
    wid = lax.axis_index("s") * _NC + lax.axis_index("c")
    wbase = wid * _BPW * _SEQ

    def chunk_body(g, carry):
        pltpu.sync_copy(xt_hbm.at[pl.ds(wbase + g * _CH, _CH)], idx_v)
        pltpu.async_copy(t_hbm.at[idx_v], vals_v, sem).wait()

        def seq_body(s, accs):
            return tuple(
                accs[k] + vals_v[pl.ds((k * _SEQ + s) * _L, _L)]
                for k in range(_NG))

        accs = lax.fori_loop(
            0, _SEQ, seq_body,
            tuple(jnp.zeros((_L,), jnp.float32) for _ in range(_NG)))
        for k in range(_NG):
            z = accs[k] * (1.0 / _SEQ)
            outb_v[pl.ds(g * _EPC + k * _L, _L)] = 1.0 / (1.0 + jnp.exp(-z))
        return carry

    lax.fori_loop(0, _NCHUNK, chunk_body, 0)
    pltpu.sync_copy(outb_v, out_hbm.at[pl.ds(wid * _BPW, _BPW)])


def kernel(x, table, W, b):
    t = _matvec(table, W.reshape(1, _DIM), b)
    xt_flat = x.reshape(_SEQ, _BATCH // _L, _L).transpose(1, 0, 2).reshape(-1)
    return _sc_pool(t, xt_flat)


# speedup vs baseline: 3.3652x; 1.1916x over previous
import functools

import jax
import jax.numpy as jnp
from jax import lax
from jax.experimental import pallas as pl
from jax.experimental.pallas import tpu as pltpu
import jax.experimental.pallas.tpu_sc as plsc

_VOCAB = 1000000
_DIM = 64
_SEQ = 200
_BATCH = 16384


_BLK = 16384
_GRID = -(-_VOCAB // _BLK)


def _matvec_body(tb_ref, w_ref, b_ref, out_ref):
    tb = tb_ref[...]
    w = w_ref[...]
    z = jax.lax.dot_general(w, tb, (((1,), (1,)), ((), ())),
                            preferred_element_type=jnp.float32)
    out_ref[...] = z[0] + b_ref[0]


def _matvec(table, w_row, b):
    return pl.pallas_call(
        _matvec_body,
        grid=(_GRID,),
        in_specs=[
            pl.BlockSpec((_BLK, _DIM), lambda i: (i, 0)),
            pl.BlockSpec((1, _DIM), lambda i: (0, 0)),
            pl.BlockSpec(memory_space=pltpu.SMEM),
        ],
        out_specs=pl.BlockSpec((_BLK,), lambda i: (i,)),
        out_shape=jax.ShapeDtypeStruct((_VOCAB,), jnp.float32),
    )(table, w_row, b)


_NC, _NS, _L = 2, 16, 16
_NW = _NC * _NS
_BPW = _BATCH // _NW
_PW = 128
_NPASS = _BPW // _PW
_NG = _PW // _L


@functools.partial(
    pl.kernel,
    out_type=jax.ShapeDtypeStruct((_BATCH,), jnp.float32),
    mesh=plsc.VectorSubcoreMesh(core_axis_name="c", subcore_axis_name="s"),
    scratch_types=[
        pltpu.VMEM((_SEQ, _PW), jnp.int32),
        pltpu.VMEM((_SEQ, _PW), jnp.float32),
        pltpu.VMEM((_BPW,), jnp.float32),
        pltpu.SemaphoreType.DMA,
    ],
)
def _sc_pool(t_hbm, x_hbm, out_hbm, idx_v, vals_v, outb_v, sem):
    wid = lax.axis_index("s") * _NC + lax.axis_index("c")
    base = wid * _BPW

    def pass_body(p, carry):
        col = base + p * _PW
        pltpu.sync_copy(x_hbm.at[pl.ds(0, _SEQ), pl.ds(col, _PW)], idx_v)

        def fire(s, carry2):
            pltpu.async_copy(t_hbm.at[idx_v.at[s]], vals_v.at[s], sem)
            return carry2

        lax.fori_loop(0, _SEQ, fire, 0)

        def drain(s, carry2):
            pltpu.make_async_copy(t_hbm.at[idx_v.at[0]], vals_v.at[0],
                                  sem).wait()
            return carry2

        lax.fori_loop(0, _SEQ, drain, 0)

        def seq_body(s, accs):
            return tuple(accs[k] + vals_v[s, pl.ds(k * _L, _L)]
                         for k in range(_NG))

        accs = lax.fori_loop(
            0, _SEQ, seq_body,
            tuple(jnp.zeros((_L,), jnp.float32) for _ in range(_NG)))
        for k in range(_NG):
            z = accs[k] * (1.0 / _SEQ)
            outb_v[pl.ds(p * _PW + k * _L, _L)] = 1.0 / (1.0 + jnp.exp(-z))
        return carry

    lax.fori_loop(0, _NPASS, pass_body, 0)
    pltpu.sync_copy(outb_v, out_hbm.at[pl.ds(base, _BPW)])


def kernel(x, table, W, b):
    t = _matvec(table, W.reshape(1, _DIM), b)
    return _sc_pool(t, x)


# speedup vs baseline: 9.1343x; 2.7144x over previous
import functools

import jax
import jax.numpy as jnp
from jax import lax
from jax.experimental import pallas as pl
from jax.experimental.pallas import tpu as pltpu
import jax.experimental.pallas.tpu_sc as plsc

_VOCAB = 1000000
_DIM = 64
_SEQ = 200
_BATCH = 16384


_BLK = 16384
_GRID = -(-_VOCAB // _BLK)


def _matvec_body(tt_ref, w_ref, b_ref, out_ref):
    tt = tt_ref[...]
    w = w_ref[...]
    z = jnp.dot(w, tt, preferred_element_type=jnp.float32)
    out_ref[...] = z[0] + b_ref[0]


def _matvec(table_t, w_row, b):
    return pl.pallas_call(
        _matvec_body,
        grid=(_GRID,),
        in_specs=[
            pl.BlockSpec((_DIM, _BLK), lambda i: (0, i)),
            pl.BlockSpec((1, _DIM), lambda i: (0, 0)),
            pl.BlockSpec(memory_space=pltpu.SMEM),
        ],
        out_specs=pl.BlockSpec((_BLK,), lambda i: (i,)),
        out_shape=jax.ShapeDtypeStruct((_VOCAB,), jnp.float32),
    )(table_t, w_row, b)


_NC, _NS, _L = 2, 16, 16
_NW = _NC * _NS
_BPW = _BATCH // _NW
_PW = 128
_NPASS = _BPW // _PW
_NG = _PW // _L


@functools.partial(
    pl.kernel,
    out_type=jax.ShapeDtypeStruct((_BATCH,), jnp.float32),
    mesh=plsc.VectorSubcoreMesh(core_axis_name="c", subcore_axis_name="s"),
    scratch_types=[
        pltpu.VMEM((_SEQ, _PW), jnp.int32),
        pltpu.VMEM((_SEQ, _PW), jnp.float32),
        pltpu.VMEM((_BPW,), jnp.float32),
        pltpu.SemaphoreType.DMA,
    ],
)
def _sc_pool(t_hbm, x_hbm, out_hbm, idx_v, vals_v, outb_v, sem):
    wid = lax.axis_index("s") * _NC + lax.axis_index("c")
    base = wid * _BPW

    def pass_body(p, carry):
        col = base + p * _PW
        pltpu.sync_copy(x_hbm.at[pl.ds(0, _SEQ), pl.ds(col, _PW)], idx_v)

        def fire(s, carry2):
            pltpu.async_copy(t_hbm.at[idx_v.at[s]], vals_v.at[s], sem)
            return carry2

        lax.fori_loop(0, _SEQ, fire, 0)

        def drain(s, carry2):
            pltpu.make_async_copy(t_hbm.at[idx_v.at[0]], vals_v.at[0],
                                  sem).wait()
            return carry2

        lax.fori_loop(0, _SEQ, drain, 0)

        def seq_body(s, accs):
            return tuple(accs[k] + vals_v[s, pl.ds(k * _L, _L)]
                         for k in range(_NG))

        accs = lax.fori_loop(
            0, _SEQ, seq_body,
            tuple(jnp.zeros((_L,), jnp.float32) for _ in range(_NG)))
        for k in range(_NG):
            z = accs[k] * (1.0 / _SEQ)
            outb_v[pl.ds(p * _PW + k * _L, _L)] = 1.0 / (1.0 + jnp.exp(-z))
        return carry

    lax.fori_loop(0, _NPASS, pass_body, 0)
    pltpu.sync_copy(outb_v, out_hbm.at[pl.ds(base, _BPW)])


def kernel(x, table, W, b):
    t = _matvec(table.T, W.reshape(1, _DIM), b)
    return _sc_pool(t, x)


# speedup vs baseline: 13.7148x; 1.5015x over previous
import functools

import jax
import jax.numpy as jnp
from jax import lax
from jax.experimental import pallas as pl
from jax.experimental.pallas import tpu as pltpu
import jax.experimental.pallas.tpu_sc as plsc

_VOCAB = 1000000
_DIM = 64
_SEQ = 200
_BATCH = 16384


_BLK = 16384
_GRID = -(-_VOCAB // _BLK)


def _matvec_body(tt_ref, w_ref, b_ref, out_ref):
    tt = tt_ref[...]
    w = w_ref[...]
    z = jnp.dot(w, tt, preferred_element_type=jnp.float32)
    out_ref[...] = z[0] + b_ref[0]


def _matvec(table_t, w_row, b):
    return pl.pallas_call(
        _matvec_body,
        grid=(_GRID,),
        in_specs=[
            pl.BlockSpec((_DIM, _BLK), lambda i: (0, i)),
            pl.BlockSpec((1, _DIM), lambda i: (0, 0)),
            pl.BlockSpec(memory_space=pltpu.SMEM),
        ],
        out_specs=pl.BlockSpec((_BLK,), lambda i: (i,)),
        out_shape=jax.ShapeDtypeStruct((_VOCAB,), jnp.float32),
    )(table_t, w_row, b)


_NC, _NS, _L = 2, 16, 16
_NW = _NC * _NS
_BPW = _BATCH // _NW
_PW = 128
_NPASS = _BPW // _PW
_NG = _PW // _L


@functools.partial(
    pl.kernel,
    out_type=jax.ShapeDtypeStruct((_BATCH,), jnp.float32),
    mesh=plsc.VectorSubcoreMesh(core_axis_name="c", subcore_axis_name="s"),
    scratch_types=[
        pltpu.VMEM((_SEQ, _PW), jnp.int32),
        pltpu.VMEM((_SEQ, _PW), jnp.float32),
        pltpu.VMEM((_BPW,), jnp.float32),
        pltpu.VMEM((10000,), jnp.float32),
        pltpu.VMEM_SHARED((_VOCAB,), jnp.float32),
        pltpu.SemaphoreType.DMA,
    ],
)
def _sc_pool(t_hbm, x_hbm, out_hbm, idx_v, vals_v, outb_v, stage_v, t_sh,
             sem):
    wid = lax.axis_index("s") * _NC + lax.axis_index("c")
    base = wid * _BPW

    sid = lax.axis_index("s")
    _C = 10000
    _NCH = _VOCAB // _C

    for j in range(-(-_NCH // _NS)):
        cid = sid + _NS * j

        @pl.when(cid < _NCH)
        def _stage(cid=cid):
            pltpu.sync_copy(t_hbm.at[pl.ds(cid * _C, _C)], stage_v)
            pltpu.sync_copy(stage_v, t_sh.at[pl.ds(cid * _C, _C)])

    plsc.subcore_barrier()

    def pass_body(p, carry):
        col = base + p * _PW
        pltpu.sync_copy(x_hbm.at[pl.ds(0, _SEQ), pl.ds(col, _PW)], idx_v)

        def fire(s, carry2):
            pltpu.async_copy(t_sh.at[idx_v.at[s]], vals_v.at[s], sem)
            return carry2

        lax.fori_loop(0, _SEQ, fire, 0)

        def drain(s, carry2):
            pltpu.make_async_copy(t_hbm.at[idx_v.at[0]], vals_v.at[0],
                                  sem).wait()
            return carry2

        lax.fori_loop(0, _SEQ, drain, 0)

        def seq_body(s, accs):
            return tuple(accs[k] + vals_v[s, pl.ds(k * _L, _L)]
                         for k in range(_NG))

        accs = lax.fori_loop(
            0, _SEQ, seq_body,
            tuple(jnp.zeros((_L,), jnp.float32) for _ in range(_NG)))
        for k in range(_NG):
            z = accs[k] * (1.0 / _SEQ)
            outb_v[pl.ds(p * _PW + k * _L, _L)] = 1.0 / (1.0 + jnp.exp(-z))
        return carry

    lax.fori_loop(0, _NPASS, pass_body, 0)
    pltpu.sync_copy(outb_v, out_hbm.at[pl.ds(base, _BPW)])


def kernel(x, table, W, b):
    t = _matvec(table.T, W.reshape(1, _DIM), b)
    return _sc_pool(t, x)
